# Initial kernel scaffold; baseline (speedup 1.0000x reference)
import jax
import jax.numpy as jnp
from jax.experimental import pallas as pl


def kernel(x, edge_index, edge_attr, W0, b0, W1, b1, W2, b2):
    raise NotImplementedError("write your pallas kernel here")


# speedup vs baseline: 4.4070x; 4.4070x over previous
import functools

import jax
import jax.numpy as jnp
from jax import lax
from jax.experimental import pallas as pl
from jax.experimental.pallas import tpu as pltpu
from jax.experimental.pallas import tpu_sc as plsc

NC = 2
NS = 16
NW = NC * NS
LANES = 16
C = 128


def _cdiv(a, b):
  return (a + b - 1) // b


def _make_sc_scatter(mode, N, NP, D, ET, ECH):
  rows_per_tile = NP // NS
  zrows = 32

  mesh = plsc.VectorSubcoreMesh(
      core_axis_name="c", subcore_axis_name="s", num_cores=NC,
      num_subcores=NS)

  @functools.partial(
      pl.kernel,
      mesh=mesh,
      out_type=jax.ShapeDtypeStruct((NC, NP, D), jnp.float32),
      scratch_types=[
          pltpu.VMEM((C,), jnp.int32),
          pltpu.VMEM((C,), jnp.int32),
          pltpu.VMEM((C,), jnp.int32),
          pltpu.VMEM((1, C), jnp.int32),
          pltpu.VMEM((1, C), jnp.int32),
          pltpu.VMEM((C, D), jnp.float32),
          pltpu.VMEM((zrows, D), jnp.float32),
          pltpu.VMEM_SHARED((NP, D), jnp.float32),
          pltpu.SemaphoreType.DMA,
      ],
  )
  def sc_kernel(src_hbm, dst_hbm, attr_hbm, table_hbm, out_hbm,
                edg_s, edg_d, edg_a, gidx, sidx, rows, zbuf, acc, sem):
    cid = lax.axis_index("c")
    sid = lax.axis_index("s")
    wid = cid * NS + sid

    zv = jnp.zeros((LANES,), jnp.float32)

    def zfill(r, carry):
      for i in range(D // LANES):
        zbuf[r, pl.ds(i * LANES, LANES)] = zv
      return carry

    lax.fori_loop(0, zrows, zfill, 0)
    for t in range(rows_per_tile // zrows):
      pltpu.sync_copy(zbuf, acc.at[pl.ds(sid * rows_per_tile + t * zrows,
                                         zrows)])
    plsc.subcore_barrier()

    base = wid * ET

    def step(j, carry):
      off = base + j * C
      pltpu.sync_copy(src_hbm.at[pl.ds(off, C)], edg_s)
      pltpu.sync_copy(dst_hbm.at[pl.ds(off, C)], edg_d)
      pltpu.sync_copy(attr_hbm.at[pl.ds(off, C)], edg_a)
      for i in range(C // LANES):
        s = edg_s[pl.ds(i * LANES, LANES)]
        d = edg_d[pl.ds(i * LANES, LANES)]
        a = edg_a[pl.ds(i * LANES, LANES)]
        if mode == 0:
          g = s
          dv = jnp.where(a == 1, d, N)
        else:
          g = s + jnp.where(a == 2, NP, 0)
          dv = jnp.where(a >= 1, d, N)
        gidx[0, pl.ds(i * LANES, LANES)] = g
        sidx[0, pl.ds(i * LANES, LANES)] = dv
      pltpu.async_copy(table_hbm.at[gidx.at[0]], rows, sem).wait()
      pltpu.sync_copy(rows, acc.at[sidx.at[0]], add=True)
      return carry

    lax.fori_loop(0, ECH, step, 0)
    plsc.subcore_barrier()

    pltpu.sync_copy(acc.at[pl.ds(sid * rows_per_tile, rows_per_tile)],
                    out_hbm.at[cid, pl.ds(sid * rows_per_tile,
                                          rows_per_tile)])

  return sc_kernel


def _mm_kernel(x, W, b, BM=1024):
  NP, D = x.shape

  def body(x_ref, w_ref, b_ref, o_ref):
    o_ref[...] = jnp.dot(x_ref[...], w_ref[...],
                         preferred_element_type=jnp.float32) + b_ref[...]

  return pl.pallas_call(
      body,
      grid=(NP // BM,),
      in_specs=[
          pl.BlockSpec((BM, D), lambda i: (i, 0)),
          pl.BlockSpec((D, D), lambda i: (0, 0)),
          pl.BlockSpec((1, D), lambda i: (0, 0)),
      ],
      out_specs=pl.BlockSpec((BM, D), lambda i: (i, 0)),
      out_shape=jax.ShapeDtypeStruct((NP, D), jnp.float32),
  )(x, W, b)


def _update_mm2_kernel(x, p, W1, b1, W2, b2, BM=1024):
  NP, D = x.shape

  def body(x_ref, p_ref, w1_ref, b1_ref, w2_ref, b2_ref,
           x1_ref, h1_ref, h2_ref):
    a = p_ref[0] + p_ref[1]
    x1 = x_ref[...] + jnp.maximum(a, 0.0)
    nrm = jnp.sqrt(jnp.sum(x1 * x1, axis=1, keepdims=True))
    x1 = x1 / jnp.maximum(nrm, 1e-12)
    x1_ref[...] = x1
    h1_ref[...] = jnp.dot(x1, w1_ref[...],
                          preferred_element_type=jnp.float32) + b1_ref[...]
    h2_ref[...] = jnp.dot(x1, w2_ref[...],
                          preferred_element_type=jnp.float32) + b2_ref[...]

  return pl.pallas_call(
      body,
      grid=(NP // BM,),
      in_specs=[
          pl.BlockSpec((BM, D), lambda i: (i, 0)),
          pl.BlockSpec((2, BM, D), lambda i: (0, i, 0)),
          pl.BlockSpec((D, D), lambda i: (0, 0)),
          pl.BlockSpec((1, D), lambda i: (0, 0)),
          pl.BlockSpec((D, D), lambda i: (0, 0)),
          pl.BlockSpec((1, D), lambda i: (0, 0)),
      ],
      out_specs=[
          pl.BlockSpec((BM, D), lambda i: (i, 0)),
          pl.BlockSpec((BM, D), lambda i: (i, 0)),
          pl.BlockSpec((BM, D), lambda i: (i, 0)),
      ],
      out_shape=[
          jax.ShapeDtypeStruct((NP, D), jnp.float32),
          jax.ShapeDtypeStruct((NP, D), jnp.float32),
          jax.ShapeDtypeStruct((NP, D), jnp.float32),
      ],
  )(x, p, W1, b1, W2, b2)


def _update_kernel(x, p, BM=1024):
  NP, D = x.shape

  def body(x_ref, p_ref, o_ref):
    a = p_ref[0] + p_ref[1]
    x1 = x_ref[...] + jnp.maximum(a, 0.0)
    nrm = jnp.sqrt(jnp.sum(x1 * x1, axis=1, keepdims=True))
    o_ref[...] = x1 / jnp.maximum(nrm, 1e-12)

  return pl.pallas_call(
      body,
      grid=(NP // BM,),
      in_specs=[
          pl.BlockSpec((BM, D), lambda i: (i, 0)),
          pl.BlockSpec((2, BM, D), lambda i: (0, i, 0)),
      ],
      out_specs=pl.BlockSpec((BM, D), lambda i: (i, 0)),
      out_shape=jax.ShapeDtypeStruct((NP, D), jnp.float32),
  )(x, p)


def kernel(x, edge_index, edge_attr, W0, b0, W1, b1, W2, b2):
  N, D = x.shape
  E = edge_index.shape[1]

  NP = _cdiv(N, 1024) * 1024
  EP = _cdiv(E, NW * C) * (NW * C)
  ET = EP // NW
  ECH = ET // C

  src = jnp.pad(edge_index[0], (0, EP - E))
  dst = jnp.pad(edge_index[1], (0, EP - E))
  attr = jnp.pad(edge_attr, (0, EP - E))
  xp = jnp.pad(x, ((0, NP - N), (0, 0)))
  b0r = b0.reshape(1, D)
  b1r = b1.reshape(1, D)
  b2r = b2.reshape(1, D)

  sc0 = _make_sc_scatter(0, N, NP, D, ET, ECH)
  sc1 = _make_sc_scatter(1, N, NP, D, ET, ECH)

  h0 = _mm_kernel(xp, W0, b0r)
  p0 = sc0(src, dst, attr, h0)
  x1, h1, h2 = _update_mm2_kernel(xp, p0, W1, b1r, W2, b2r)
  h12 = jnp.concatenate([h1, h2], axis=0)
  p1 = sc1(src, dst, attr, h12)
  out = _update_kernel(x1, p1)
  return out[:N]
